# baseline (device time: 166737 ns/iter reference)
import jax
import jax.numpy as jnp
from jax import lax
from jax.experimental import pallas as pl
from jax.experimental.pallas import tpu as pltpu

N_DEV = 8

QCLIP = 5.5
QSCALE = 127.0 / QCLIP


def kernel(A, B):
    m_per, k = A.shape
    _, n = B.shape

    def body(a_ref, b_ref, out_ref, aq_ref, comm, bbf, c_ref,
             send_sems, recv_sems, copy_sems):
        my = lax.axis_index("i")

        def sig(p):
            return jnp.where(p < 4, p, 11 - p)

        idx = sig(my)

        barrier_sem = pltpu.get_barrier_semaphore()
        for j in range(1, N_DEV):
            pl.semaphore_signal(
                barrier_sem, inc=1,
                device_id=(lax.rem(my + j, N_DEV),),
                device_id_type=pl.DeviceIdType.MESH,
            )
        pl.semaphore_wait(barrier_sem, N_DEV - 1)

        aq_ref[...] = jnp.clip(
            jnp.round(a_ref[...] * QSCALE), -127.0, 127.0
        ).astype(jnp.int8)
        bbf[...] = (b_ref[...] * (1.0 / QSCALE)).astype(jnp.bfloat16)

        sent = []
        for j in range(1, N_DEV):
            rdma = pltpu.make_async_remote_copy(
                src_ref=aq_ref,
                dst_ref=comm.at[my],
                send_sem=send_sems.at[j - 1],
                recv_sem=recv_sems.at[my],
                device_id=(lax.rem(my + j, N_DEV),),
                device_id_type=pl.DeviceIdType.MESH,
            )
            rdma.start()
            sent.append(rdma)

        state = {"i": 0, "pending": {}}

        def store_block(origin, chunk_ref):
            s = state["i"] % 2
            if s in state["pending"]:
                state["pending"][s].wait()
            c_ref[s] = jnp.dot(
                chunk_ref[...].astype(jnp.bfloat16), bbf[...],
                preferred_element_type=jnp.float32,
            )
            cp = pltpu.make_async_copy(
                c_ref.at[s],
                out_ref.at[pl.ds(origin * m_per, m_per), :],
                copy_sems.at[state["i"]],
            )
            cp.start()
            state["pending"][s] = cp
            state["i"] += 1

        store_block(my, aq_ref)

        offsets = [1, -1, 2, -2, 3, -3, 4]
        for off in offsets:
            origin = sig(lax.rem(idx + off + N_DEV, N_DEV))
            recv = pltpu.make_async_remote_copy(
                src_ref=comm.at[origin],
                dst_ref=comm.at[origin],
                send_sem=send_sems.at[0],
                recv_sem=recv_sems.at[origin],
                device_id=(origin,),
                device_id_type=pl.DeviceIdType.MESH,
            )
            recv.wait_recv()
            store_block(origin, comm.at[origin])

        for rdma in sent:
            rdma.wait_send()
        for cp in state["pending"].values():
            cp.wait()

    return pl.pallas_call(
        body,
        out_shape=jax.ShapeDtypeStruct((N_DEV * m_per, n), jnp.float32),
        in_specs=[
            pl.BlockSpec(memory_space=pltpu.VMEM),
            pl.BlockSpec(memory_space=pltpu.VMEM),
        ],
        out_specs=pl.BlockSpec(memory_space=pltpu.MemorySpace.HBM),
        scratch_shapes=[
            pltpu.VMEM((m_per, k), jnp.int8),
            pltpu.VMEM((N_DEV, m_per, k), jnp.int8),
            pltpu.VMEM((k, n), jnp.bfloat16),
            pltpu.VMEM((2, m_per, n), jnp.float32),
            pltpu.SemaphoreType.DMA((N_DEV - 1,)),
            pltpu.SemaphoreType.DMA((N_DEV,)),
            pltpu.SemaphoreType.DMA((N_DEV,)),
        ],
        compiler_params=pltpu.CompilerParams(
            collective_id=0,
            vmem_limit_bytes=100 * 1024 * 1024,
        ),
    )(A, B)


# device time: 117951 ns/iter; 1.4136x vs baseline; 1.4136x over previous
import jax
import jax.numpy as jnp
from jax import lax
from jax.experimental import pallas as pl
from jax.experimental.pallas import tpu as pltpu

N_DEV = 8
R_HOPS = N_DEV // 2
L_HOPS = N_DEV - 1 - R_HOPS

QCLIP = 5.5
QSCALE = 127.0 / QCLIP


def kernel(A, B):
    m_per, k = A.shape
    _, n = B.shape

    def body(a_ref, b_ref, out_ref, rcomm, lcomm, bbf, c_ref,
             r_send, r_recv, l_send, l_recv, copy_sems):
        my = lax.axis_index("i")

        def sig(p):
            return jnp.where(p < 4, p, 11 - p)

        idx = sig(my)
        right = sig(lax.rem(idx + 1, N_DEV))
        left = sig(lax.rem(idx + N_DEV - 1, N_DEV))

        barrier_sem = pltpu.get_barrier_semaphore()
        for nbr in (left, right):
            pl.semaphore_signal(
                barrier_sem, inc=1,
                device_id=(nbr,), device_id_type=pl.DeviceIdType.MESH,
            )
        pl.semaphore_wait(barrier_sem, 2)

        a_q = jnp.clip(
            jnp.round(a_ref[...] * QSCALE), -127.0, 127.0
        ).astype(jnp.int8)
        rcomm[0] = a_q
        lcomm[0] = a_q
        bbf[...] = (b_ref[...] * (1.0 / QSCALE)).astype(jnp.bfloat16)

        state = {"i": 0, "pending": {}}

        def store_block(origin, chunk_ref):
            s = state["i"] % 2
            if s in state["pending"]:
                state["pending"][s].wait()
            c_ref[s] = jnp.dot(
                chunk_ref[...].astype(jnp.bfloat16), bbf[...],
                preferred_element_type=jnp.float32,
            )
            cp = pltpu.make_async_copy(
                c_ref.at[s],
                out_ref.at[pl.ds(origin * m_per, m_per), :],
                copy_sems.at[state["i"]],
            )
            cp.start()
            state["pending"][s] = cp
            state["i"] += 1

        hm = m_per // 2

        def r_rdma(h, j):
            return pltpu.make_async_remote_copy(
                src_ref=rcomm.at[h, pl.ds(j * hm, hm), :],
                dst_ref=rcomm.at[h + 1, pl.ds(j * hm, hm), :],
                send_sem=r_send.at[2 * h + j], recv_sem=r_recv.at[2 * h + j],
                device_id=(right,), device_id_type=pl.DeviceIdType.MESH,
            )

        def l_rdma(h, j):
            return pltpu.make_async_remote_copy(
                src_ref=lcomm.at[h, pl.ds(j * hm, hm), :],
                dst_ref=lcomm.at[h + 1, pl.ds(j * hm, hm), :],
                send_sem=l_send.at[2 * h + j], recv_sem=l_recv.at[2 * h + j],
                device_id=(left,), device_id_type=pl.DeviceIdType.MESH,
            )

        sent = []
        rprev = [None, None]
        lprev = [None, None]
        for j in (0, 1):
            rprev[j] = r_rdma(0, j)
            rprev[j].start()
            lprev[j] = l_rdma(0, j)
            lprev[j].start()
            sent += [rprev[j], lprev[j]]
        store_block(my, rcomm.at[0])

        for h in range(1, R_HOPS + 1):
            for j in (0, 1):
                rprev[j].wait_recv()
                if h < R_HOPS:
                    rprev[j] = r_rdma(h, j)
                    rprev[j].start()
                    sent.append(rprev[j])
                if h - 1 < L_HOPS:
                    lprev[j].wait_recv()
                    if h < L_HOPS:
                        lprev[j] = l_rdma(h, j)
                        lprev[j].start()
                        sent.append(lprev[j])
            store_block(sig(lax.rem(idx - h + N_DEV, N_DEV)), rcomm.at[h])
            if h <= L_HOPS:
                store_block(sig(lax.rem(idx + h, N_DEV)), lcomm.at[h])
        for rdma in sent:
            rdma.wait_send()
        for cp in state["pending"].values():
            cp.wait()

    return pl.pallas_call(
        body,
        out_shape=jax.ShapeDtypeStruct((N_DEV * m_per, n), jnp.float32),
        in_specs=[
            pl.BlockSpec(memory_space=pltpu.VMEM),
            pl.BlockSpec(memory_space=pltpu.VMEM),
        ],
        out_specs=pl.BlockSpec(memory_space=pltpu.MemorySpace.HBM),
        scratch_shapes=[
            pltpu.VMEM((R_HOPS + 1, m_per, k), jnp.int8),
            pltpu.VMEM((L_HOPS + 1, m_per, k), jnp.int8),
            pltpu.VMEM((k, n), jnp.bfloat16),
            pltpu.VMEM((2, m_per, n), jnp.float32),
            pltpu.SemaphoreType.DMA((2 * R_HOPS,)),
            pltpu.SemaphoreType.DMA((2 * R_HOPS,)),
            pltpu.SemaphoreType.DMA((2 * L_HOPS,)),
            pltpu.SemaphoreType.DMA((2 * L_HOPS,)),
            pltpu.SemaphoreType.DMA((N_DEV,)),
        ],
        compiler_params=pltpu.CompilerParams(
            collective_id=0,
            vmem_limit_bytes=100 * 1024 * 1024,
        ),
    )(A, B)


# device time: 114702 ns/iter; 1.4537x vs baseline; 1.0283x over previous
import jax
import jax.numpy as jnp
from jax import lax
from jax.experimental import pallas as pl
from jax.experimental.pallas import tpu as pltpu

N_DEV = 8
R_HOPS = N_DEV // 2
L_HOPS = N_DEV - 1 - R_HOPS

QCLIP = 5.5
QSCALE = 127.0 / QCLIP


def kernel(A, B):
    m_per, k = A.shape
    _, n = B.shape

    def body(a_ref, b_ref, out_ref, rcomm, lcomm, bbf, c_ref,
             r_send, r_recv, l_send, l_recv, copy_sems):
        my = lax.axis_index("i")

        def sig(p):
            return jnp.where(p < 4, p, 11 - p)

        idx = sig(my)
        right = sig(lax.rem(idx + 1, N_DEV))
        left = sig(lax.rem(idx + N_DEV - 1, N_DEV))

        barrier_sem = pltpu.get_barrier_semaphore()
        for nbr in (left, right):
            pl.semaphore_signal(
                barrier_sem, inc=1,
                device_id=(nbr,), device_id_type=pl.DeviceIdType.MESH,
            )
        pl.semaphore_wait(barrier_sem, 2)

        state = {"i": 0, "pending": {}}

        def store_block(origin, chunk_ref):
            s = state["i"] % 2
            if s in state["pending"]:
                state["pending"][s].wait()
            c_ref[s] = jnp.dot(
                chunk_ref[...].astype(jnp.bfloat16), bbf[...],
                preferred_element_type=jnp.float32,
            )
            cp = pltpu.make_async_copy(
                c_ref.at[s],
                out_ref.at[pl.ds(origin * m_per, m_per), :],
                copy_sems.at[state["i"]],
            )
            cp.start()
            state["pending"][s] = cp
            state["i"] += 1

        hm = m_per // 2

        def r_rdma(h, j):
            return pltpu.make_async_remote_copy(
                src_ref=rcomm.at[h, pl.ds(j * hm, hm), :],
                dst_ref=rcomm.at[h + 1, pl.ds(j * hm, hm), :],
                send_sem=r_send.at[2 * h + j], recv_sem=r_recv.at[2 * h + j],
                device_id=(right,), device_id_type=pl.DeviceIdType.MESH,
            )

        def l_rdma(h, j):
            src = rcomm if h == 0 else lcomm
            return pltpu.make_async_remote_copy(
                src_ref=src.at[0 if h == 0 else h, pl.ds(j * hm, hm), :],
                dst_ref=lcomm.at[h + 1, pl.ds(j * hm, hm), :],
                send_sem=l_send.at[2 * h + j], recv_sem=l_recv.at[2 * h + j],
                device_id=(left,), device_id_type=pl.DeviceIdType.MESH,
            )

        sent = []
        rprev = [None, None]
        lprev = [None, None]
        for j in (0, 1):
            rows = pl.ds(j * hm, hm)
            rcomm[0, rows, :] = jnp.clip(
                jnp.round(a_ref[rows, :] * QSCALE), -127.0, 127.0
            ).astype(jnp.int8)
            rprev[j] = r_rdma(0, j)
            rprev[j].start()
            lprev[j] = l_rdma(0, j)
            lprev[j].start()
            sent += [rprev[j], lprev[j]]
        bbf[...] = (b_ref[...] * (1.0 / QSCALE)).astype(jnp.bfloat16)
        store_block(my, rcomm.at[0])

        for h in range(1, R_HOPS):
            for j in (0, 1):
                rprev[j].wait_recv()
                rprev[j] = r_rdma(h, j)
                rprev[j].start()
                sent.append(rprev[j])
                if h - 1 < L_HOPS:
                    lprev[j].wait_recv()
                    if h < L_HOPS:
                        lprev[j] = l_rdma(h, j)
                        lprev[j].start()
                        sent.append(lprev[j])
            store_block(sig(lax.rem(idx - h + N_DEV, N_DEV)), rcomm.at[h])
            if h <= L_HOPS:
                store_block(sig(lax.rem(idx + h, N_DEV)), lcomm.at[h])

        origin = sig(lax.rem(idx - R_HOPS + N_DEV, N_DEV))
        s = state["i"] % 2
        if s in state["pending"]:
            state["pending"][s].wait()
            del state["pending"][s]
        for j in (0, 1):
            rows = pl.ds(j * hm, hm)
            rprev[j].wait_recv()
            c_ref[s, rows, :] = jnp.dot(
                rcomm[R_HOPS, rows, :].astype(jnp.bfloat16), bbf[...],
                preferred_element_type=jnp.float32,
            )
            cp = pltpu.make_async_copy(
                c_ref.at[s, rows, :],
                out_ref.at[pl.ds(origin * m_per + j * hm, hm), :],
                copy_sems.at[state["i"]],
            )
            cp.start()
            state["pending"][10 + j] = cp
            state["i"] += 1

        for rdma in sent:
            rdma.wait_send()
        for cp in state["pending"].values():
            cp.wait()

    return pl.pallas_call(
        body,
        out_shape=jax.ShapeDtypeStruct((N_DEV * m_per, n), jnp.float32),
        in_specs=[
            pl.BlockSpec(memory_space=pltpu.VMEM),
            pl.BlockSpec(memory_space=pltpu.VMEM),
        ],
        out_specs=pl.BlockSpec(memory_space=pltpu.MemorySpace.HBM),
        scratch_shapes=[
            pltpu.VMEM((R_HOPS + 1, m_per, k), jnp.int8),
            pltpu.VMEM((L_HOPS + 1, m_per, k), jnp.int8),
            pltpu.VMEM((k, n), jnp.bfloat16),
            pltpu.VMEM((2, m_per, n), jnp.float32),
            pltpu.SemaphoreType.DMA((2 * R_HOPS,)),
            pltpu.SemaphoreType.DMA((2 * R_HOPS,)),
            pltpu.SemaphoreType.DMA((2 * L_HOPS,)),
            pltpu.SemaphoreType.DMA((2 * L_HOPS,)),
            pltpu.SemaphoreType.DMA((N_DEV + 1,)),
        ],
        compiler_params=pltpu.CompilerParams(
            collective_id=0,
            vmem_limit_bytes=100 * 1024 * 1024,
        ),
    )(A, B)


# device time: 112978 ns/iter; 1.4758x vs baseline; 1.0153x over previous
import jax
import jax.numpy as jnp
from jax import lax
from jax.experimental import pallas as pl
from jax.experimental.pallas import tpu as pltpu

N_DEV = 8
R_HOPS = N_DEV // 2
L_HOPS = N_DEV - 1 - R_HOPS

QCLIP = 5.5
QSCALE = 127.0 / QCLIP
SUB = 4


def kernel(A, B):
    m_per, k = A.shape
    _, n = B.shape

    def body(a_ref, b_ref, out_ref, rcomm, lcomm, bbf, c_ref,
             r_send, r_recv, l_send, l_recv, copy_sems):
        my = lax.axis_index("i")

        def sig(p):
            return jnp.where(p < 4, p, 11 - p)

        idx = sig(my)
        right = sig(lax.rem(idx + 1, N_DEV))
        left = sig(lax.rem(idx + N_DEV - 1, N_DEV))

        barrier_sem = pltpu.get_barrier_semaphore()
        for nbr in (left, right):
            pl.semaphore_signal(
                barrier_sem, inc=1,
                device_id=(nbr,), device_id_type=pl.DeviceIdType.MESH,
            )
        pl.semaphore_wait(barrier_sem, 2)

        state = {"i": 0, "pending": {}}

        def store_block(origin, chunk_ref):
            s = state["i"] % 2
            if s in state["pending"]:
                state["pending"][s].wait()
            c_ref[s] = jnp.dot(
                chunk_ref[...].astype(jnp.bfloat16), bbf[...],
                preferred_element_type=jnp.float32,
            )
            cp = pltpu.make_async_copy(
                c_ref.at[s],
                out_ref.at[pl.ds(origin * m_per, m_per), :],
                copy_sems.at[state["i"]],
            )
            cp.start()
            state["pending"][s] = cp
            state["i"] += 1

        hm = m_per // SUB

        def r_rdma(h, j):
            return pltpu.make_async_remote_copy(
                src_ref=rcomm.at[h, pl.ds(j * hm, hm), :],
                dst_ref=rcomm.at[h + 1, pl.ds(j * hm, hm), :],
                send_sem=r_send.at[SUB * h + j], recv_sem=r_recv.at[SUB * h + j],
                device_id=(right,), device_id_type=pl.DeviceIdType.MESH,
            )

        def l_rdma(h, j):
            src = rcomm if h == 0 else lcomm
            return pltpu.make_async_remote_copy(
                src_ref=src.at[0 if h == 0 else h, pl.ds(j * hm, hm), :],
                dst_ref=lcomm.at[h + 1, pl.ds(j * hm, hm), :],
                send_sem=l_send.at[SUB * h + j], recv_sem=l_recv.at[SUB * h + j],
                device_id=(left,), device_id_type=pl.DeviceIdType.MESH,
            )

        sent = []
        rprev = [None] * SUB
        lprev = [None] * SUB
        for j in range(SUB):
            rows = pl.ds(j * hm, hm)
            rcomm[0, rows, :] = jnp.clip(
                jnp.round(a_ref[rows, :] * QSCALE), -127.0, 127.0
            ).astype(jnp.int8)
            rprev[j] = r_rdma(0, j)
            rprev[j].start()
            lprev[j] = l_rdma(0, j)
            lprev[j].start()
            sent += [rprev[j], lprev[j]]
        bbf[...] = (b_ref[...] * (1.0 / QSCALE)).astype(jnp.bfloat16)
        store_block(my, rcomm.at[0])

        for h in range(1, R_HOPS):
            for j in range(SUB):
                rprev[j].wait_recv()
                rprev[j] = r_rdma(h, j)
                rprev[j].start()
                sent.append(rprev[j])
                if h - 1 < L_HOPS:
                    lprev[j].wait_recv()
                    if h < L_HOPS:
                        lprev[j] = l_rdma(h, j)
                        lprev[j].start()
                        sent.append(lprev[j])
            store_block(sig(lax.rem(idx - h + N_DEV, N_DEV)), rcomm.at[h])
            if h <= L_HOPS:
                store_block(sig(lax.rem(idx + h, N_DEV)), lcomm.at[h])

        origin = sig(lax.rem(idx - R_HOPS + N_DEV, N_DEV))
        s = state["i"] % 2
        if s in state["pending"]:
            state["pending"][s].wait()
            del state["pending"][s]
        for j in range(SUB):
            rows = pl.ds(j * hm, hm)
            rprev[j].wait_recv()
            c_ref[s, rows, :] = jnp.dot(
                rcomm[R_HOPS, rows, :].astype(jnp.bfloat16), bbf[...],
                preferred_element_type=jnp.float32,
            )
            cp = pltpu.make_async_copy(
                c_ref.at[s, rows, :],
                out_ref.at[pl.ds(origin * m_per + j * hm, hm), :],
                copy_sems.at[state["i"]],
            )
            cp.start()
            state["pending"][10 + j] = cp
            state["i"] += 1

        for rdma in sent:
            rdma.wait_send()
        for cp in state["pending"].values():
            cp.wait()

    return pl.pallas_call(
        body,
        out_shape=jax.ShapeDtypeStruct((N_DEV * m_per, n), jnp.float32),
        in_specs=[
            pl.BlockSpec(memory_space=pltpu.VMEM),
            pl.BlockSpec(memory_space=pltpu.VMEM),
        ],
        out_specs=pl.BlockSpec(memory_space=pltpu.MemorySpace.HBM),
        scratch_shapes=[
            pltpu.VMEM((R_HOPS + 1, m_per, k), jnp.int8),
            pltpu.VMEM((L_HOPS + 1, m_per, k), jnp.int8),
            pltpu.VMEM((k, n), jnp.bfloat16),
            pltpu.VMEM((2, m_per, n), jnp.float32),
            pltpu.SemaphoreType.DMA((SUB * R_HOPS,)),
            pltpu.SemaphoreType.DMA((SUB * R_HOPS,)),
            pltpu.SemaphoreType.DMA((SUB * L_HOPS,)),
            pltpu.SemaphoreType.DMA((SUB * L_HOPS,)),
            pltpu.SemaphoreType.DMA((N_DEV + SUB,)),
        ],
        compiler_params=pltpu.CompilerParams(
            collective_id=0,
            vmem_limit_bytes=100 * 1024 * 1024,
        ),
    )(A, B)
